# baseline (speedup 1.0000x reference)
import functools

import jax
import jax.numpy as jnp
from jax import lax
from jax.experimental import pallas as pl
from jax.experimental.pallas import tpu as pltpu
from jax.experimental.pallas import tpu_sc as plsc

B, N, E, H, NSTEP, NLAYERS = 8, 1250, 320000, 128, 3, 2
BN = B * N
NC, NS, L = 2, 16, 16
NW = NC * NS


def _ln_body(x_ref, g_ref, b_ref, o_ref):
    x = x_ref[...]
    m = jnp.mean(x, axis=-1, keepdims=True)
    v = jnp.mean((x - m) ** 2, axis=-1, keepdims=True)
    o_ref[...] = (x - m) * lax.rsqrt(v + 1e-5) * g_ref[...] + b_ref[...]


def _ln_pallas(x2d, g, b):
    rows = x2d.shape[0]
    blk = 1000
    return pl.pallas_call(
        _ln_body,
        grid=(rows // blk,),
        in_specs=[
            pl.BlockSpec((blk, H), lambda i: (i, 0)),
            pl.BlockSpec((1, H), lambda i: (0, 0)),
            pl.BlockSpec((1, H), lambda i: (0, 0)),
        ],
        out_specs=pl.BlockSpec((blk, H), lambda i: (i, 0)),
        out_shape=jax.ShapeDtypeStruct((rows, H), jnp.float32),
    )(x2d, g.reshape(1, H), b.reshape(1, H))


_GEC = 128
_GCHUNKS = E // _GEC


def _sc_gather_body(ent_hbm, pk_hbm, tprod_hbm,
                    pk_a, pk_b, hrow_a, trow_a, hrow_b, trow_b,
                    pks_a, pks_b, gs_a, gs_b, ws_a, ws_b):
    cid = lax.axis_index("c")
    sid = lax.axis_index("s")
    wid = sid * NC + cid
    n_my = (_GCHUNKS - wid + NW - 1) // NW
    slots = ((pk_a, hrow_a, trow_a, pks_a, gs_a, ws_a),
             (pk_b, hrow_b, trow_b, pks_b, gs_b, ws_b))

    def pk_start(k, s):
        pltpu.async_copy(pk_hbm.at[wid + k * NW], slots[s][0], slots[s][3])

    def pk_wait(s):
        pltpu.make_async_copy(pk_hbm.at[0], slots[s][0], slots[s][3]).wait()

    def g_start(s):
        pk, hr, tr, _, gs, _ = slots[s]
        pltpu.async_copy(ent_hbm.at[pk.at[0]], hr, gs)
        pltpu.async_copy(ent_hbm.at[pk.at[1]], tr, gs)

    def g_wait(s):
        pk, hr, tr, _, gs, _ = slots[s]
        pltpu.make_async_copy(ent_hbm.at[pk.at[0]], hr, gs).wait()
        pltpu.make_async_copy(ent_hbm.at[pk.at[1]], tr, gs).wait()

    def w_start(k, s):
        base = (wid + k * NW) * _GEC
        _, hr, _, _, _, ws = slots[s]
        pltpu.async_copy(hr, tprod_hbm.at[pl.ds(base, _GEC), :], ws)

    def w_wait(s):
        _, hr, _, _, _, ws = slots[s]
        pltpu.make_async_copy(hr, tprod_hbm.at[pl.ds(0, _GEC), :], ws).wait()

    pk_start(0, 0)
    pk_start(1, 1)
    pk_wait(0)
    g_start(0)

    def pair(j, carry):
        for b in (0, 1):
            k = 2 * j + b

            @pl.when(k < n_my)
            def _():
                _, hr, tr, _, _, _ = slots[b]
                g_wait(b)

                @pl.when(k + 1 < n_my)
                def _():
                    pk_wait(1 - b)

                    @pl.when(k >= 1)
                    def _():
                        w_wait(1 - b)
                    g_start(1 - b)

                @plsc.parallel_loop(0, _GEC, 1, unroll=8)
                def _prod(e):
                    for h in range(H // L):
                        hv = hr[e, pl.ds(h * L, L)]
                        tv = tr[e, pl.ds(h * L, L)]
                        hr[e, pl.ds(h * L, L)] = hv * tv

                w_start(k, b)

                @pl.when(k + 2 < n_my)
                def _():
                    pk_start(k + 2, b)
        return carry

    lax.fori_loop(0, (n_my + 1) // 2, pair, 0)
    w_wait(0)
    w_wait(1)


_sc_gather = functools.partial(
    pl.kernel,
    out_type=[
        jax.ShapeDtypeStruct((E, H), jnp.float32),
    ],
    mesh=plsc.VectorSubcoreMesh(core_axis_name="c", subcore_axis_name="s"),
    compiler_params=pltpu.CompilerParams(needs_layout_passes=False),
    scratch_types=[
        pltpu.VMEM((3, 128), jnp.int32),
        pltpu.VMEM((3, 128), jnp.int32),
        pltpu.VMEM((_GEC, H), jnp.float32),
        pltpu.VMEM((_GEC, H), jnp.float32),
        pltpu.VMEM((_GEC, H), jnp.float32),
        pltpu.VMEM((_GEC, H), jnp.float32),
        pltpu.SemaphoreType.DMA,
        pltpu.SemaphoreType.DMA,
        pltpu.SemaphoreType.DMA,
        pltpu.SemaphoreType.DMA,
        pltpu.SemaphoreType.DMA,
        pltpu.SemaphoreType.DMA,
    ],
)(_sc_gather_body)


_KBLK = 2000
_KGRID = E // _KBLK


def _kge_body(fr_ref, tp_ref, w_ref, b_ref, o_ref):
    i = pl.program_id(0)
    r = jnp.dot(fr_ref[...], w_ref[...], preferred_element_type=jnp.float32)
    r = r + b_ref[...]
    s = jnp.sum(tp_ref[...] * r, axis=-1)
    p = jnp.maximum(-s, 0.0) + jnp.log1p(jnp.exp(-jnp.abs(s)))
    part = jnp.full((1, 1), jnp.sum(p))

    @pl.when(i == 0)
    def _():
        o_ref[...] = jnp.zeros((1, 1), jnp.float32)

    o_ref[...] += part

    @pl.when(i == _KGRID - 1)
    def _():
        o_ref[...] = o_ref[...] * (1.0 / E)


def _kge_pallas(fr, tp, rel_W, rel_b):
    return pl.pallas_call(
        _kge_body,
        grid=(_KGRID,),
        in_specs=[
            pl.BlockSpec((_KBLK, H), lambda i: (i, 0)),
            pl.BlockSpec((_KBLK, H), lambda i: (i, 0)),
            pl.BlockSpec((H, H), lambda i: (0, 0)),
            pl.BlockSpec((1, H), lambda i: (0, 0)),
        ],
        out_specs=pl.BlockSpec((1, 1), lambda i: (0, 0)),
        out_shape=jax.ShapeDtypeStruct((1, 1), jnp.float32),
    )(fr, tp, rel_W, rel_b.reshape(1, H))


_SEC = 128
_SROWS = _SEC // 128
_SCHUNKS = E // _SEC
_ZR = 128


def _sc_step_body(fr_hbm, pk_hbm, prior_hbm, q_hbm,
                  out_hbm, mout_hbm,
                  acc_sh, macc_sh,
                  prior_v, q_v, fr_a, fr_b, pk_a, pk_b, pv_a, pv_b,
                  ls_a, ls_b, ss_a, ss_b):
    cid = lax.axis_index("c")
    sid = lax.axis_index("s")
    wid = sid * NC + cid
    slots = ((fr_a, pk_a, pv_a, ls_a, ss_a),
             (fr_b, pk_b, pv_b, ls_b, ss_b))

    def zrow(r, carry):
        for h in range(H // L):
            fr_a[r, pl.ds(h * L, L)] = jnp.zeros((L,), jnp.float32)
        return carry

    lax.fori_loop(0, _ZR, zrow, 0)
    base_z = jnp.minimum(sid * 625 - (sid % 8), BN - 5 * _ZR)
    for k in range(5):
        pltpu.sync_copy(fr_a.at[pl.ds(0, _ZR), :],
                        acc_sh.at[pl.ds(base_z + k * _ZR, _ZR), :])

    @pl.when(sid == 0)
    def _():
        def mzrow(r, carry):
            prior_v[pl.ds(r * L, L)] = jnp.zeros((L,), jnp.float32)
            return carry
        lax.fori_loop(0, BN // L, mzrow, 0)
        pltpu.sync_copy(prior_v, macc_sh)

    pltpu.sync_copy(prior_hbm, prior_v)
    pltpu.sync_copy(q_hbm, q_v)
    plsc.subcore_barrier()

    n_my = (_SCHUNKS - wid + NW - 1) // NW

    def l_start(k, s):
        frs, pks, _, ls, _ = slots[s]
        c = wid + k * NW
        pltpu.async_copy(fr_hbm.at[pl.ds(c * _SEC, _SEC), :], frs, ls)
        pltpu.async_copy(pk_hbm.at[c], pks, ls)

    def l_wait(s):
        frs, pks, _, ls, _ = slots[s]
        pltpu.make_async_copy(fr_hbm.at[pl.ds(0, _SEC), :], frs, ls).wait()
        pltpu.make_async_copy(pk_hbm.at[0], pks, ls).wait()

    def s_start(s):
        frs, pks, pvs, _, ss = slots[s]
        pltpu.async_copy(frs, acc_sh.at[pks.at[1]], ss, add=True)
        pltpu.async_copy(pvs, macc_sh.at[pks.at[1]], ss, add=True)

    def s_wait(s):
        frs, pks, pvs, _, ss = slots[s]
        pltpu.make_async_copy(frs, acc_sh.at[pks.at[1]], ss).wait()
        pltpu.make_async_copy(pvs, macc_sh.at[pks.at[1]], ss).wait()

    l_start(0, 0)

    def pair(j, carry):
        for b in (0, 1):
            k = 2 * j + b

            @pl.when(k < n_my)
            def _():
                frs, pks, pvs, _, _ = slots[b]
                l_wait(b)

                for j2 in range(_SEC // L):
                    h16 = pks[0, pl.ds(j2 * L, L)]
                    pvs[pl.ds(j2 * L, L)] = plsc.load_gather(prior_v, [h16])

                @pl.when(k >= 1)
                def _():
                    s_wait(1 - b)

                @pl.when(k + 1 < n_my)
                def _():
                    l_start(k + 1, 1 - b)

                r2 = jnp.full((L,), 2, jnp.int32)
                hcols = [lax.iota(jnp.int32, L) + (h * L)
                         for h in range(H // L)]

                @plsc.parallel_loop(0, _SEC, 1, unroll=16)
                def _edge(e):
                    esp = jnp.full((L,), e, jnp.int32)
                    pvec = plsc.load_gather(pvs, [esp])
                    bvec = plsc.load_gather(pks, [r2, esp])
                    for h in range(H // L):
                        qv = plsc.load_gather(q_v, [bvec, hcols[h]])
                        frv = frs[e, pl.ds(h * L, L)]
                        frs[e, pl.ds(h * L, L)] = (
                            pvec * jnp.maximum(qv * frv, 0.0))

                s_start(b)
        return carry

    lax.fori_loop(0, (n_my + 1) // 2, pair, 0)
    @pl.when(n_my % 2 == 1)
    def _():
        s_wait(0)

    @pl.when(n_my % 2 == 0)
    def _():
        s_wait(1)

    plsc.subcore_barrier()

    @pl.when(sid == 0)
    def _():
        pltpu.sync_copy(acc_sh, out_hbm.at[cid])
        pltpu.sync_copy(macc_sh, mout_hbm.at[cid])


_sc_step = functools.partial(
    pl.kernel,
    out_type=[
        jax.ShapeDtypeStruct((NC, BN, H), jnp.float32),
        jax.ShapeDtypeStruct((NC, BN), jnp.float32),
    ],
    mesh=plsc.VectorSubcoreMesh(core_axis_name="c", subcore_axis_name="s"),
    compiler_params=pltpu.CompilerParams(needs_layout_passes=False),
    scratch_types=[
        pltpu.VMEM_SHARED((BN, H), jnp.float32),
        pltpu.VMEM_SHARED((BN,), jnp.float32),
        pltpu.VMEM((BN,), jnp.float32),
        pltpu.VMEM((B, H), jnp.float32),
        pltpu.VMEM((_SEC, H), jnp.float32),
        pltpu.VMEM((_SEC, H), jnp.float32),
        pltpu.VMEM((3, 128), jnp.int32),
        pltpu.VMEM((3, 128), jnp.int32),
        pltpu.VMEM((_SEC,), jnp.float32),
        pltpu.VMEM((_SEC,), jnp.float32),
        pltpu.SemaphoreType.DMA,
        pltpu.SemaphoreType.DMA,
        pltpu.SemaphoreType.DMA,
        pltpu.SemaphoreType.DMA,
    ],
)(_sc_step_body)


def _step_tc_body(part_ref, mpart_ref, prior_ref, emask_ref,
                  h0_ref, c0_ref, h1_ref, c1_ref,
                  ih0_ref, hh0_ref, hb0_ref, ih1_ref, hh1_ref, hb1_ref,
                  g_ref, b_ref, sw_ref, sb_ref, fw_ref, fb_ref,
                  nh0_ref, nc0_ref, nh1_ref, nc1_ref, lab_ref, fin_ref):
    g = g_ref[...]
    bb = b_ref[...]

    def ln(x):
        m = jnp.mean(x, axis=-1, keepdims=True)
        v = jnp.mean((x - m) ** 2, axis=-1, keepdims=True)
        return (x - m) * lax.rsqrt(v + 1e-5) * g + bb

    neighbor = part_ref[0, 0] + part_ref[1, 0]
    imask = mpart_ref[0, 0] + mpart_ref[1, 0]
    lh = ln(neighbor)

    states = ((h0_ref[0], c0_ref[0], ih0_ref, hh0_ref, hb0_ref,
               nh0_ref, nc0_ref),
              (h1_ref[0], c1_ref[0], ih1_ref, hh1_ref, hb1_ref,
               nh1_ref, nc1_ref))
    for (h_prev, c_prev, ihw, hhw, hb, nh_ref, nc_ref) in states:
        z = (jnp.dot(lh, ihw[...], preferred_element_type=jnp.float32)
             + jnp.dot(h_prev, hhw[...], preferred_element_type=jnp.float32)
             + hb[...])
        ig = jax.nn.sigmoid(z[:, 0:H])
        fg = jax.nn.sigmoid(z[:, H:2 * H])
        og = jax.nn.sigmoid(z[:, 2 * H:3 * H])
        mg = jnp.tanh(z[:, 3 * H:4 * H])
        c = fg * c_prev + ig * mg
        hh = og * jnp.tanh(c)
        hh = ln(hh)
        c = ln(c)
        nh_ref[0] = hh
        nc_ref[0] = c
        lh = hh

    m = (imask + prior_ref[0] > 1e-8).astype(jnp.float32)
    m = emask_ref[0] * m
    s = (jnp.dot(lh, sw_ref[...], preferred_element_type=jnp.float32)
         + sb_ref[...])
    s = m * s + (1.0 - m) * -1e20
    smax = jnp.max(s, axis=0, keepdims=True)
    es = jnp.exp(s - smax)
    lab_ref[0] = es / jnp.sum(es, axis=0, keepdims=True)

    fin_ref[0] = (jnp.dot(lh, fw_ref[...], preferred_element_type=jnp.float32)
                  + fb_ref[...])


def _step_tc(part, mpart, prior, emask, h0, c0, h1, c1,
             ih0_W, hh0_W, hh0_b, ih1_W, hh1_W, hh1_b,
             ln_g, ln_b, score_W, score_b, ffn_W, ffn_b):
    snh = jax.ShapeDtypeStruct((B, N, H), jnp.float32)
    full = lambda shape: pl.BlockSpec(shape, lambda i: tuple(0 for _ in shape))
    bnh = pl.BlockSpec((1, N, H), lambda i: (i, 0, 0))
    bn1 = pl.BlockSpec((1, N, 1), lambda i: (i, 0, 0))
    return pl.pallas_call(
        _step_tc_body,
        grid=(B,),
        in_specs=[
            pl.BlockSpec((2, 1, N, H), lambda i: (0, i, 0, 0)),
            pl.BlockSpec((2, 1, N, 1), lambda i: (0, i, 0, 0)),
            bn1, bn1, bnh, bnh, bnh, bnh,
            full((H, 4 * H)), full((H, 4 * H)), full((1, 4 * H)),
            full((H, 4 * H)), full((H, 4 * H)), full((1, 4 * H)),
            full((1, H)), full((1, H)), full((H, 1)), full((1, 1)),
            full((H, H)), full((1, H)),
        ],
        out_specs=[bnh, bnh, bnh, bnh, bn1, bnh],
        out_shape=[snh, snh, snh, snh,
                   jax.ShapeDtypeStruct((B, N, 1), jnp.float32), snh],
    )(part, mpart, prior, emask, h0, c0, h1, c1,
      ih0_W, hh0_W, hh0_b.reshape(1, 4 * H),
      ih1_W, hh1_W, hh1_b.reshape(1, 4 * H),
      ln_g.reshape(1, H), ln_b.reshape(1, H),
      score_W, score_b.reshape(1, 1), ffn_W, ffn_b.reshape(1, H))


def kernel(instructions, entity_emb, fact_relations, topic_label, entity_mask,
           batch_ids, head_idx, tail_idx,
           rel_W, rel_b, ih0_W, hh0_W, hh0_b, ih1_W, hh1_W, hh1_b,
           ln_g, ln_b, score_W, score_b, ffn_W, ffn_b):
    ent2d = _ln_pallas(entity_emb.reshape(BN, H), ln_g, ln_b)
    ent = ent2d.reshape(B, N, H)

    pk = jnp.stack([head_idx.reshape(E // 128, 128),
                    tail_idx.reshape(E // 128, 128),
                    batch_ids.reshape(E // 128, 128)], axis=1)

    (tp,) = _sc_gather(ent2d, pk)
    kge = _kge_pallas(fact_relations, tp, rel_W, rel_b)[0, 0]

    emask3 = entity_mask.reshape(B, N, 1)

    prior = topic_label.reshape(BN)
    h0 = c0 = h1 = c1 = ent
    labels = []
    fin = None
    for i in range(NSTEP):
        part, mpart = _sc_step(fact_relations, pk, prior, instructions[i])
        h0, c0, h1, c1, lab, fin = _step_tc(
            part.reshape(NC, B, N, H), mpart.reshape(NC, B, N, 1),
            prior.reshape(B, N, 1), emask3, h0, c0, h1, c1,
            ih0_W, hh0_W, hh0_b, ih1_W, hh1_W, hh1_b,
            ln_g, ln_b, score_W, score_b, ffn_W, ffn_b)
        labels.append(lab.reshape(B, N))
        prior = lab.reshape(BN)

    return (jnp.stack(labels, axis=0), fin, kge)


# speedup vs baseline: 1.0326x; 1.0326x over previous
import functools

import jax
import jax.numpy as jnp
from jax import lax
from jax.experimental import pallas as pl
from jax.experimental.pallas import tpu as pltpu
from jax.experimental.pallas import tpu_sc as plsc

B, N, E, H, NSTEP, NLAYERS = 8, 1250, 320000, 128, 3, 2
BN = B * N
NC, NS, L = 2, 16, 16
NW = NC * NS


def _ln_body(x_ref, g_ref, b_ref, o_ref):
    x = x_ref[...]
    m = jnp.mean(x, axis=-1, keepdims=True)
    v = jnp.mean((x - m) ** 2, axis=-1, keepdims=True)
    o_ref[...] = (x - m) * lax.rsqrt(v + 1e-5) * g_ref[...] + b_ref[...]


def _ln_pallas(x2d, g, b):
    rows = x2d.shape[0]
    blk = 1000
    return pl.pallas_call(
        _ln_body,
        grid=(rows // blk,),
        in_specs=[
            pl.BlockSpec((blk, H), lambda i: (i, 0)),
            pl.BlockSpec((1, H), lambda i: (0, 0)),
            pl.BlockSpec((1, H), lambda i: (0, 0)),
        ],
        out_specs=pl.BlockSpec((blk, H), lambda i: (i, 0)),
        out_shape=jax.ShapeDtypeStruct((rows, H), jnp.float32),
    )(x2d, g.reshape(1, H), b.reshape(1, H))


_GEC = 128
_GCHUNKS = E // _GEC


def _sc_gather_body(ent_hbm, pk_hbm, tprod_hbm,
                    pk_a, pk_b, pk_c, hrow_a, trow_a, hrow_b, trow_b,
                    hrow_c, trow_c,
                    pks_a, pks_b, pks_c, gs_a, gs_b, gs_c, ws_a, ws_b, ws_c):
    cid = lax.axis_index("c")
    sid = lax.axis_index("s")
    wid = sid * NC + cid
    n_my = (_GCHUNKS - wid + NW - 1) // NW
    slots = ((pk_a, hrow_a, trow_a, pks_a, gs_a, ws_a),
             (pk_b, hrow_b, trow_b, pks_b, gs_b, ws_b),
             (pk_c, hrow_c, trow_c, pks_c, gs_c, ws_c))

    def pk_start(k, s):
        pltpu.async_copy(pk_hbm.at[wid + k * NW], slots[s][0], slots[s][3])

    def pk_wait(s):
        pltpu.make_async_copy(pk_hbm.at[0], slots[s][0], slots[s][3]).wait()

    def g_start(s):
        pk, hr, tr, _, gs, _ = slots[s]
        pltpu.async_copy(ent_hbm.at[pk.at[0]], hr, gs)
        pltpu.async_copy(ent_hbm.at[pk.at[1]], tr, gs)

    def g_wait(s):
        pk, hr, tr, _, gs, _ = slots[s]
        pltpu.make_async_copy(ent_hbm.at[pk.at[0]], hr, gs).wait()
        pltpu.make_async_copy(ent_hbm.at[pk.at[1]], tr, gs).wait()

    def w_start(k, s):
        base = (wid + k * NW) * _GEC
        _, hr, _, _, _, ws = slots[s]
        pltpu.async_copy(hr, tprod_hbm.at[pl.ds(base, _GEC), :], ws)

    def w_wait(s):
        _, hr, _, _, _, ws = slots[s]
        pltpu.make_async_copy(hr, tprod_hbm.at[pl.ds(0, _GEC), :], ws).wait()

    pk_start(0, 0)
    pk_start(1, 1)
    pk_start(2, 2)
    pk_wait(0)
    g_start(0)
    pk_wait(1)
    g_start(1)

    def triple(j, carry):
        for b in (0, 1, 2):
            k = 3 * j + b

            @pl.when(k < n_my)
            def _():
                _, hr, tr, _, _, _ = slots[b]
                g_wait(b)

                @pl.when(k + 3 < n_my)
                def _():
                    pk_start(k + 3, b)

                s2 = (b + 2) % 3

                @pl.when(k + 2 < n_my)
                def _():
                    pk_wait(s2)

                    @pl.when(k >= 1)
                    def _():
                        w_wait(s2)
                    g_start(s2)

                @plsc.parallel_loop(0, _GEC, 1, unroll=8)
                def _prod(e):
                    for h in range(H // L):
                        hv = hr[e, pl.ds(h * L, L)]
                        tv = tr[e, pl.ds(h * L, L)]
                        hr[e, pl.ds(h * L, L)] = hv * tv

                w_start(k, b)
        return carry

    lax.fori_loop(0, (n_my + 2) // 3, triple, 0)
    w_wait(0)
    w_wait(1)
    w_wait(2)


_sc_gather = functools.partial(
    pl.kernel,
    out_type=[
        jax.ShapeDtypeStruct((E, H), jnp.float32),
    ],
    mesh=plsc.VectorSubcoreMesh(core_axis_name="c", subcore_axis_name="s"),
    compiler_params=pltpu.CompilerParams(needs_layout_passes=False),
    scratch_types=[
        pltpu.VMEM((3, 128), jnp.int32),
        pltpu.VMEM((3, 128), jnp.int32),
        pltpu.VMEM((3, 128), jnp.int32),
        pltpu.VMEM((_GEC, H), jnp.float32),
        pltpu.VMEM((_GEC, H), jnp.float32),
        pltpu.VMEM((_GEC, H), jnp.float32),
        pltpu.VMEM((_GEC, H), jnp.float32),
        pltpu.VMEM((_GEC, H), jnp.float32),
        pltpu.VMEM((_GEC, H), jnp.float32),
        pltpu.SemaphoreType.DMA,
        pltpu.SemaphoreType.DMA,
        pltpu.SemaphoreType.DMA,
        pltpu.SemaphoreType.DMA,
        pltpu.SemaphoreType.DMA,
        pltpu.SemaphoreType.DMA,
        pltpu.SemaphoreType.DMA,
        pltpu.SemaphoreType.DMA,
        pltpu.SemaphoreType.DMA,
    ],
)(_sc_gather_body)


_KBLK = 2000
_KGRID = E // _KBLK


def _kge_body(fr_ref, tp_ref, w_ref, b_ref, o_ref):
    i = pl.program_id(0)
    r = jnp.dot(fr_ref[...], w_ref[...], preferred_element_type=jnp.float32)
    r = r + b_ref[...]
    s = jnp.sum(tp_ref[...] * r, axis=-1)
    p = jnp.maximum(-s, 0.0) + jnp.log1p(jnp.exp(-jnp.abs(s)))
    part = jnp.full((1, 1), jnp.sum(p))

    @pl.when(i == 0)
    def _():
        o_ref[...] = jnp.zeros((1, 1), jnp.float32)

    o_ref[...] += part

    @pl.when(i == _KGRID - 1)
    def _():
        o_ref[...] = o_ref[...] * (1.0 / E)


def _kge_pallas(fr, tp, rel_W, rel_b):
    return pl.pallas_call(
        _kge_body,
        grid=(_KGRID,),
        in_specs=[
            pl.BlockSpec((_KBLK, H), lambda i: (i, 0)),
            pl.BlockSpec((_KBLK, H), lambda i: (i, 0)),
            pl.BlockSpec((H, H), lambda i: (0, 0)),
            pl.BlockSpec((1, H), lambda i: (0, 0)),
        ],
        out_specs=pl.BlockSpec((1, 1), lambda i: (0, 0)),
        out_shape=jax.ShapeDtypeStruct((1, 1), jnp.float32),
    )(fr, tp, rel_W, rel_b.reshape(1, H))


_SEC = 128
_SROWS = _SEC // 128
_SCHUNKS = E // _SEC
_ZR = 128


def _sc_step_body(fr_hbm, pk_hbm, prior_hbm, q_hbm,
                  out_hbm, mout_hbm,
                  acc_sh, macc_sh,
                  prior_v, q_v, fr_a, fr_b, pk_a, pk_b, pv_a, pv_b,
                  ls_a, ls_b, ss_a, ss_b):
    cid = lax.axis_index("c")
    sid = lax.axis_index("s")
    wid = sid * NC + cid
    slots = ((fr_a, pk_a, pv_a, ls_a, ss_a),
             (fr_b, pk_b, pv_b, ls_b, ss_b))

    def zrow(r, carry):
        for h in range(H // L):
            fr_a[r, pl.ds(h * L, L)] = jnp.zeros((L,), jnp.float32)
        return carry

    lax.fori_loop(0, _ZR, zrow, 0)
    base_z = jnp.minimum(sid * 625 - (sid % 8), BN - 5 * _ZR)
    for k in range(5):
        pltpu.sync_copy(fr_a.at[pl.ds(0, _ZR), :],
                        acc_sh.at[pl.ds(base_z + k * _ZR, _ZR), :])

    @pl.when(sid == 0)
    def _():
        def mzrow(r, carry):
            prior_v[pl.ds(r * L, L)] = jnp.zeros((L,), jnp.float32)
            return carry
        lax.fori_loop(0, BN // L, mzrow, 0)
        pltpu.sync_copy(prior_v, macc_sh)

    pltpu.sync_copy(prior_hbm, prior_v)
    pltpu.sync_copy(q_hbm, q_v)
    plsc.subcore_barrier()

    n_my = (_SCHUNKS - wid + NW - 1) // NW

    def l_start(k, s):
        frs, pks, _, ls, _ = slots[s]
        c = wid + k * NW
        pltpu.async_copy(fr_hbm.at[pl.ds(c * _SEC, _SEC), :], frs, ls)
        pltpu.async_copy(pk_hbm.at[c], pks, ls)

    def l_wait(s):
        frs, pks, _, ls, _ = slots[s]
        pltpu.make_async_copy(fr_hbm.at[pl.ds(0, _SEC), :], frs, ls).wait()
        pltpu.make_async_copy(pk_hbm.at[0], pks, ls).wait()

    def s_start(s):
        frs, pks, pvs, _, ss = slots[s]
        pltpu.async_copy(frs, acc_sh.at[pks.at[1]], ss, add=True)
        pltpu.async_copy(pvs, macc_sh.at[pks.at[1]], ss, add=True)

    def s_wait(s):
        frs, pks, pvs, _, ss = slots[s]
        pltpu.make_async_copy(frs, acc_sh.at[pks.at[1]], ss).wait()
        pltpu.make_async_copy(pvs, macc_sh.at[pks.at[1]], ss).wait()

    l_start(0, 0)

    def pair(j, carry):
        for b in (0, 1):
            k = 2 * j + b

            @pl.when(k < n_my)
            def _():
                frs, pks, pvs, _, _ = slots[b]
                l_wait(b)

                for j2 in range(_SEC // L):
                    h16 = pks[0, pl.ds(j2 * L, L)]
                    pvs[pl.ds(j2 * L, L)] = plsc.load_gather(prior_v, [h16])

                @pl.when(k >= 1)
                def _():
                    s_wait(1 - b)

                @pl.when(k + 1 < n_my)
                def _():
                    l_start(k + 1, 1 - b)

                r2 = jnp.full((L,), 2, jnp.int32)
                hcols = [lax.iota(jnp.int32, L) + (h * L)
                         for h in range(H // L)]

                @plsc.parallel_loop(0, _SEC, 1, unroll=8)
                def _edge(e):
                    esp = jnp.full((L,), e, jnp.int32)
                    pvec = plsc.load_gather(pvs, [esp])
                    bvec = plsc.load_gather(pks, [r2, esp])
                    for h in range(H // L):
                        qv = plsc.load_gather(q_v, [bvec, hcols[h]])
                        frv = frs[e, pl.ds(h * L, L)]
                        frs[e, pl.ds(h * L, L)] = (
                            pvec * jnp.maximum(qv * frv, 0.0))

                s_start(b)
        return carry

    lax.fori_loop(0, (n_my + 1) // 2, pair, 0)
    @pl.when(n_my % 2 == 1)
    def _():
        s_wait(0)

    @pl.when(n_my % 2 == 0)
    def _():
        s_wait(1)

    plsc.subcore_barrier()

    @pl.when(sid == 0)
    def _():
        pltpu.sync_copy(acc_sh, out_hbm.at[cid])
        pltpu.sync_copy(macc_sh, mout_hbm.at[cid])


_sc_step = functools.partial(
    pl.kernel,
    out_type=[
        jax.ShapeDtypeStruct((NC, BN, H), jnp.float32),
        jax.ShapeDtypeStruct((NC, BN), jnp.float32),
    ],
    mesh=plsc.VectorSubcoreMesh(core_axis_name="c", subcore_axis_name="s"),
    compiler_params=pltpu.CompilerParams(needs_layout_passes=False),
    scratch_types=[
        pltpu.VMEM_SHARED((BN, H), jnp.float32),
        pltpu.VMEM_SHARED((BN,), jnp.float32),
        pltpu.VMEM((BN,), jnp.float32),
        pltpu.VMEM((B, H), jnp.float32),
        pltpu.VMEM((_SEC, H), jnp.float32),
        pltpu.VMEM((_SEC, H), jnp.float32),
        pltpu.VMEM((3, 128), jnp.int32),
        pltpu.VMEM((3, 128), jnp.int32),
        pltpu.VMEM((_SEC,), jnp.float32),
        pltpu.VMEM((_SEC,), jnp.float32),
        pltpu.SemaphoreType.DMA,
        pltpu.SemaphoreType.DMA,
        pltpu.SemaphoreType.DMA,
        pltpu.SemaphoreType.DMA,
    ],
)(_sc_step_body)


def _step_tc_body(part_ref, mpart_ref, prior_ref, emask_ref,
                  h0_ref, c0_ref, h1_ref, c1_ref,
                  ih0_ref, hh0_ref, hb0_ref, ih1_ref, hh1_ref, hb1_ref,
                  g_ref, b_ref, sw_ref, sb_ref, fw_ref, fb_ref,
                  nh0_ref, nc0_ref, nh1_ref, nc1_ref, lab_ref, fin_ref):
    g = g_ref[...]
    bb = b_ref[...]

    def ln(x):
        m = jnp.mean(x, axis=-1, keepdims=True)
        v = jnp.mean((x - m) ** 2, axis=-1, keepdims=True)
        return (x - m) * lax.rsqrt(v + 1e-5) * g + bb

    neighbor = part_ref[0, 0] + part_ref[1, 0]
    imask = mpart_ref[0, 0] + mpart_ref[1, 0]
    lh = ln(neighbor)

    states = ((h0_ref[0], c0_ref[0], ih0_ref, hh0_ref, hb0_ref,
               nh0_ref, nc0_ref),
              (h1_ref[0], c1_ref[0], ih1_ref, hh1_ref, hb1_ref,
               nh1_ref, nc1_ref))
    for (h_prev, c_prev, ihw, hhw, hb, nh_ref, nc_ref) in states:
        z = (jnp.dot(lh, ihw[...], preferred_element_type=jnp.float32)
             + jnp.dot(h_prev, hhw[...], preferred_element_type=jnp.float32)
             + hb[...])
        ig = jax.nn.sigmoid(z[:, 0:H])
        fg = jax.nn.sigmoid(z[:, H:2 * H])
        og = jax.nn.sigmoid(z[:, 2 * H:3 * H])
        mg = jnp.tanh(z[:, 3 * H:4 * H])
        c = fg * c_prev + ig * mg
        hh = og * jnp.tanh(c)
        hh = ln(hh)
        c = ln(c)
        nh_ref[0] = hh
        nc_ref[0] = c
        lh = hh

    m = (imask + prior_ref[0] > 1e-8).astype(jnp.float32)
    m = emask_ref[0] * m
    s = (jnp.dot(lh, sw_ref[...], preferred_element_type=jnp.float32)
         + sb_ref[...])
    s = m * s + (1.0 - m) * -1e20
    smax = jnp.max(s, axis=0, keepdims=True)
    es = jnp.exp(s - smax)
    lab_ref[0] = es / jnp.sum(es, axis=0, keepdims=True)

    fin_ref[0] = (jnp.dot(lh, fw_ref[...], preferred_element_type=jnp.float32)
                  + fb_ref[...])


def _step_tc(part, mpart, prior, emask, h0, c0, h1, c1,
             ih0_W, hh0_W, hh0_b, ih1_W, hh1_W, hh1_b,
             ln_g, ln_b, score_W, score_b, ffn_W, ffn_b):
    snh = jax.ShapeDtypeStruct((B, N, H), jnp.float32)
    full = lambda shape: pl.BlockSpec(shape, lambda i: tuple(0 for _ in shape))
    bnh = pl.BlockSpec((1, N, H), lambda i: (i, 0, 0))
    bn1 = pl.BlockSpec((1, N, 1), lambda i: (i, 0, 0))
    return pl.pallas_call(
        _step_tc_body,
        grid=(B,),
        in_specs=[
            pl.BlockSpec((2, 1, N, H), lambda i: (0, i, 0, 0)),
            pl.BlockSpec((2, 1, N, 1), lambda i: (0, i, 0, 0)),
            bn1, bn1, bnh, bnh, bnh, bnh,
            full((H, 4 * H)), full((H, 4 * H)), full((1, 4 * H)),
            full((H, 4 * H)), full((H, 4 * H)), full((1, 4 * H)),
            full((1, H)), full((1, H)), full((H, 1)), full((1, 1)),
            full((H, H)), full((1, H)),
        ],
        out_specs=[bnh, bnh, bnh, bnh, bn1, bnh],
        out_shape=[snh, snh, snh, snh,
                   jax.ShapeDtypeStruct((B, N, 1), jnp.float32), snh],
    )(part, mpart, prior, emask, h0, c0, h1, c1,
      ih0_W, hh0_W, hh0_b.reshape(1, 4 * H),
      ih1_W, hh1_W, hh1_b.reshape(1, 4 * H),
      ln_g.reshape(1, H), ln_b.reshape(1, H),
      score_W, score_b.reshape(1, 1), ffn_W, ffn_b.reshape(1, H))


def kernel(instructions, entity_emb, fact_relations, topic_label, entity_mask,
           batch_ids, head_idx, tail_idx,
           rel_W, rel_b, ih0_W, hh0_W, hh0_b, ih1_W, hh1_W, hh1_b,
           ln_g, ln_b, score_W, score_b, ffn_W, ffn_b):
    ent2d = _ln_pallas(entity_emb.reshape(BN, H), ln_g, ln_b)
    ent = ent2d.reshape(B, N, H)

    pk = jnp.stack([head_idx.reshape(E // 128, 128),
                    tail_idx.reshape(E // 128, 128),
                    batch_ids.reshape(E // 128, 128)], axis=1)

    (tp,) = _sc_gather(ent2d, pk)
    kge = _kge_pallas(fact_relations, tp, rel_W, rel_b)[0, 0]

    emask3 = entity_mask.reshape(B, N, 1)

    prior = topic_label.reshape(BN)
    h0 = c0 = h1 = c1 = ent
    labels = []
    fin = None
    for i in range(NSTEP):
        part, mpart = _sc_step(fact_relations, pk, prior, instructions[i])
        h0, c0, h1, c1, lab, fin = _step_tc(
            part.reshape(NC, B, N, H), mpart.reshape(NC, B, N, 1),
            prior.reshape(B, N, 1), emask3, h0, c0, h1, c1,
            ih0_W, hh0_W, hh0_b, ih1_W, hh1_W, hh1_b,
            ln_g, ln_b, score_W, score_b, ffn_W, ffn_b)
        labels.append(lab.reshape(B, N))
        prior = lab.reshape(BN)

    return (jnp.stack(labels, axis=0), fin, kge)
